# Initial kernel scaffold; baseline (speedup 1.0000x reference)
import jax
import jax.numpy as jnp
from jax.experimental import pallas as pl


def kernel(x, W_enc, b_enc, W_dec0, W_dec1, W_dec2, W_dec3, W_dec4, W_dec5, W_dec6, W_dec7, b_dec0, b_dec1, b_dec2, b_dec3, b_dec4, b_dec5, b_dec6, b_dec7):
    raise NotImplementedError("write your pallas kernel here")


# speedup vs baseline: 1.4180x; 1.4180x over previous
import jax
import jax.numpy as jnp
from jax.experimental import pallas as pl
from jax.experimental.pallas import tpu as pltpu

_B = 512
_T = 8
_DIN = 768
_DSAE = 4096
_K = 128
_BASE = _DSAE // _T

_INTERPRET = False


def _encode_body(x_ref, w_ref, b_ref, z_ref, acc_ref):
    kb = pl.program_id(0)
    nk = pl.num_programs(0)

    @pl.when(kb == 0)
    def _init():
        acc_ref[...] = jnp.zeros_like(acc_ref)

    acc_ref[...] += jnp.dot(x_ref[...], w_ref[...],
                            preferred_element_type=jnp.float32)

    @pl.when(kb == nk - 1)
    def _finish():
        pre = acc_ref[...] + b_ref[...]
        u = jax.lax.bitcast_convert_type(pre, jnp.uint32)
        m = jnp.where(pre < 0.0, ~u, u | jnp.uint32(0x80000000))

        def step(i, t):
            shift = (31 - i).astype(jnp.uint32)
            cand = t | (jnp.uint32(1) << shift)
            cnt = jnp.sum((m >= cand).astype(jnp.int32), axis=1,
                          keepdims=True)
            return jnp.where(cnt >= _K, cand, t)

        t0 = jnp.zeros((pre.shape[0], 1), jnp.uint32)
        thr = jax.lax.fori_loop(0, 32, step, t0)
        z_ref[...] = jnp.where(m >= thr, jnp.maximum(pre, 0.0), 0.0)


def _encode(xf, wf, b2):
    kblk = 768
    nk = (_T * _DIN) // kblk
    return pl.pallas_call(
        _encode_body,
        grid=(nk,),
        in_specs=[
            pl.BlockSpec((_B, kblk), lambda k: (0, k)),
            pl.BlockSpec((kblk, _DSAE), lambda k: (k, 0)),
            pl.BlockSpec((1, _DSAE), lambda k: (0, 0)),
        ],
        out_specs=pl.BlockSpec((_B, _DSAE), lambda k: (0, 0)),
        out_shape=jax.ShapeDtypeStruct((_B, _DSAE), jnp.float32),
        scratch_shapes=[pltpu.VMEM((_B, _DSAE), jnp.float32)],
        compiler_params=pltpu.CompilerParams(
            vmem_limit_bytes=100 * 1024 * 1024),
        interpret=_INTERPRET,
    )(xf, wf, b2)


def _decode_body(z_ref, w_ref, b_ref, xc_ref, xhat_ref, loss_ref, acc_ref):
    j = pl.program_id(0)
    nj = pl.num_programs(0)
    out = jnp.dot(z_ref[...], w_ref[...],
                  preferred_element_type=jnp.float32) + b_ref[...]
    xhat_ref[...] = out
    d = out - xc_ref[...]
    part = jnp.sum(d * d)

    @pl.when(j == 0)
    def _init():
        acc_ref[0, 0] = 0.0

    acc_ref[0, 0] += part

    @pl.when(j == nj - 1)
    def _finish():
        loss_ref[0, 0] = acc_ref[0, 0]


def _decode(z, wf, b2, xc, s):
    prefix = _BASE * s
    n = s * _DIN
    nblk = _DIN
    xhat, loss = pl.pallas_call(
        _decode_body,
        grid=(n // nblk,),
        in_specs=[
            pl.BlockSpec((_B, prefix), lambda j: (0, 0)),
            pl.BlockSpec((prefix, nblk), lambda j: (0, j)),
            pl.BlockSpec((1, nblk), lambda j: (0, j)),
            pl.BlockSpec((_B, nblk), lambda j: (0, j)),
        ],
        out_specs=[
            pl.BlockSpec((_B, nblk), lambda j: (0, j)),
            pl.BlockSpec(memory_space=pltpu.SMEM),
        ],
        out_shape=[
            jax.ShapeDtypeStruct((_B, n), jnp.float32),
            jax.ShapeDtypeStruct((1, 1), jnp.float32),
        ],
        scratch_shapes=[pltpu.SMEM((1, 1), jnp.float32)],
        compiler_params=pltpu.CompilerParams(
            vmem_limit_bytes=100 * 1024 * 1024),
        interpret=_INTERPRET,
    )(z, wf, b2, xc)
    return xhat, loss


def kernel(x, W_enc, b_enc,
           W_dec0, W_dec1, W_dec2, W_dec3, W_dec4, W_dec5, W_dec6, W_dec7,
           b_dec0, b_dec1, b_dec2, b_dec3, b_dec4, b_dec5, b_dec6, b_dec7):
    W_decs = [W_dec0, W_dec1, W_dec2, W_dec3, W_dec4, W_dec5, W_dec6, W_dec7]
    b_decs = [b_dec0, b_dec1, b_dec2, b_dec3, b_dec4, b_dec5, b_dec6, b_dec7]

    xf = x.reshape(_B, _T * _DIN)
    wf = W_enc.reshape(_T * _DIN, _DSAE)
    z = _encode(xf, wf, b_enc.reshape(1, _DSAE))

    losses = []
    x_hat_full = None
    for t in range(_T):
        s = t + 1
        start = (_T - s) // 2
        xc = x[:, start:start + s, :].reshape(_B, s * _DIN)
        wft = W_decs[t].reshape(_BASE * s, s * _DIN)
        bft = b_decs[t].reshape(1, s * _DIN)
        xhat, loss = _decode(z, wft, bft, xc, s)
        losses.append(loss[0, 0] / (_B * s))
        if t == _T - 1:
            x_hat_full = xhat.reshape(_B, _T, _DIN)

    total_loss = jnp.mean(jnp.stack(losses))
    return total_loss, x_hat_full, z


# speedup vs baseline: 1.6812x; 1.1856x over previous
import jax
import jax.numpy as jnp
from jax.experimental import pallas as pl
from jax.experimental.pallas import tpu as pltpu

_B = 512
_T = 8
_DIN = 768
_DSAE = 4096
_K = 128
_BASE = _DSAE // _T

_INTERPRET = False


def _encode_body(x_ref, w_ref, b_ref, z_ref, zb_ref, acc_ref):
    kb = pl.program_id(0)
    nk = pl.num_programs(0)

    @pl.when(kb == 0)
    def _init():
        acc_ref[...] = jnp.zeros_like(acc_ref)

    acc_ref[...] += jnp.dot(x_ref[...], w_ref[...],
                            preferred_element_type=jnp.float32)

    @pl.when(kb == nk - 1)
    def _finish():
        pre = acc_ref[...] + b_ref[...]
        u = jax.lax.bitcast_convert_type(pre, jnp.uint32)
        m = jnp.where(pre < 0.0, ~u, u | jnp.uint32(0x80000000))

        def step(i, t):
            shift = (31 - i).astype(jnp.uint32)
            cand = t | (jnp.uint32(1) << shift)
            cnt = jnp.sum((m >= cand).astype(jnp.int32), axis=1,
                          keepdims=True)
            return jnp.where(cnt >= _K, cand, t)

        t0 = jnp.zeros((pre.shape[0], 1), jnp.uint32)
        thr = jax.lax.fori_loop(0, 32, step, t0)
        z = jnp.where(m >= thr, jnp.maximum(pre, 0.0), 0.0)
        z_ref[...] = z
        zb_ref[...] = z.astype(jnp.bfloat16)


def _encode(xf, wf, b2):
    kblk = 768
    nk = (_T * _DIN) // kblk
    return pl.pallas_call(
        _encode_body,
        grid=(nk,),
        in_specs=[
            pl.BlockSpec((_B, kblk), lambda k: (0, k)),
            pl.BlockSpec((kblk, _DSAE), lambda k: (k, 0)),
            pl.BlockSpec((1, _DSAE), lambda k: (0, 0)),
        ],
        out_specs=[
            pl.BlockSpec((_B, _DSAE), lambda k: (0, 0)),
            pl.BlockSpec((_B, _DSAE), lambda k: (0, 0)),
        ],
        out_shape=[
            jax.ShapeDtypeStruct((_B, _DSAE), jnp.float32),
            jax.ShapeDtypeStruct((_B, _DSAE), jnp.bfloat16),
        ],
        scratch_shapes=[pltpu.VMEM((_B, _DSAE), jnp.float32)],
        compiler_params=pltpu.CompilerParams(
            vmem_limit_bytes=100 * 1024 * 1024),
        interpret=_INTERPRET,
    )(xf, wf, b2)


def _decode_body(z_ref, w_ref, b_ref, xc_ref, xhat_ref, loss_ref, acc_ref):
    j = pl.program_id(0)
    nj = pl.num_programs(0)
    wb = w_ref[...].astype(jnp.bfloat16)
    out = jnp.dot(z_ref[...], wb,
                  preferred_element_type=jnp.float32) + b_ref[...]
    xhat_ref[...] = out
    d = out - xc_ref[...]
    part = jnp.sum(d * d)

    @pl.when(j == 0)
    def _init():
        acc_ref[0, 0] = 0.0

    acc_ref[0, 0] += part

    @pl.when(j == nj - 1)
    def _finish():
        loss_ref[0, 0] = acc_ref[0, 0]


def _decode(zb, wf, b2, xf, s):
    prefix = _BASE * s
    n = s * _DIN
    nblk = _DIN
    start = (_T - s) // 2
    xhat, loss = pl.pallas_call(
        _decode_body,
        grid=(n // nblk,),
        in_specs=[
            pl.BlockSpec((_B, prefix), lambda j: (0, 0)),
            pl.BlockSpec((prefix, nblk), lambda j: (0, j)),
            pl.BlockSpec((1, nblk), lambda j: (0, j)),
            pl.BlockSpec((_B, nblk), lambda j: (0, start + j)),
        ],
        out_specs=[
            pl.BlockSpec((_B, nblk), lambda j: (0, j)),
            pl.BlockSpec(memory_space=pltpu.SMEM),
        ],
        out_shape=[
            jax.ShapeDtypeStruct((_B, n), jnp.float32),
            jax.ShapeDtypeStruct((1, 1), jnp.float32),
        ],
        scratch_shapes=[pltpu.SMEM((1, 1), jnp.float32)],
        compiler_params=pltpu.CompilerParams(
            vmem_limit_bytes=100 * 1024 * 1024),
        interpret=_INTERPRET,
    )(zb, wf, b2, xf)
    return xhat, loss


def kernel(x, W_enc, b_enc,
           W_dec0, W_dec1, W_dec2, W_dec3, W_dec4, W_dec5, W_dec6, W_dec7,
           b_dec0, b_dec1, b_dec2, b_dec3, b_dec4, b_dec5, b_dec6, b_dec7):
    W_decs = [W_dec0, W_dec1, W_dec2, W_dec3, W_dec4, W_dec5, W_dec6, W_dec7]
    b_decs = [b_dec0, b_dec1, b_dec2, b_dec3, b_dec4, b_dec5, b_dec6, b_dec7]

    xf = x.reshape(_B, _T * _DIN)
    wf = W_enc.reshape(_T * _DIN, _DSAE)
    z, zb = _encode(xf, wf, b_enc.reshape(1, _DSAE))

    losses = []
    x_hat_full = None
    for t in range(_T):
        s = t + 1
        wft = W_decs[t].reshape(_BASE * s, s * _DIN)
        bft = b_decs[t].reshape(1, s * _DIN)
        xhat, loss = _decode(zb, wft, bft, xf, s)
        losses.append(loss[0, 0] / (_B * s))
        if t == _T - 1:
            x_hat_full = xhat.reshape(_B, _T, _DIN)

    total_loss = jnp.mean(jnp.stack(losses))
    return total_loss, x_hat_full, z
